# Initial kernel scaffold; baseline (speedup 1.0000x reference)
import jax
import jax.numpy as jnp
from jax.experimental import pallas as pl


def kernel(x, n_tokens, indices_s1, bin_ids_s1, bins_s1, ins_s1, outs_s1, ins_s2_virtual, outs_s2_virtual, s1exp_to_s2_indices, bin_ids_s2, bins_s2, ins_s2, outs_s2, tokens_per_expert_s1_exp, tokens_per_experts_s2_exp):
    raise NotImplementedError("write your pallas kernel here")


# speedup vs baseline: 2.6759x; 2.6759x over previous
import functools

import jax
import jax.numpy as jnp
from jax import lax
from jax.experimental import pallas as pl
from jax.experimental.pallas import tpu as pltpu
from jax.experimental.pallas import tpu_sc as plsc

_TOP_K = 2
_T = 8192
_D = 2048
_TOT = _T * _TOP_K
_B = 2 * _TOT
_NC = 2
_NS = 16
_NW = _NC * _NS
_PER_W = _B // _NW
_R = 16
_NBUF = 2
_NT = _PER_W // _R


def _dispatch_body(x_hbm, i1_hbm, s2_hbm, out_hbm,
                   idx_v, s2_v, buf0, buf1, gsem, osem):
    del osem
    bufs = (buf0, buf1)
    cid = lax.axis_index("c")
    sid = lax.axis_index("s")
    wid = sid * _NC + cid
    base = wid * _PER_W

    @pl.when(wid < _NW // 2)
    def _bottom():
        pltpu.sync_copy(s2_hbm.at[pl.ds(base, _PER_W)], s2_v)
        pltpu.async_copy(i1_hbm.at[s2_v], idx_v, gsem).wait()

    @pl.when(wid >= _NW // 2)
    def _top():
        pltpu.sync_copy(i1_hbm.at[pl.ds(base - _TOT, _PER_W)], idx_v)

    @pl.loop(0, _PER_W // 16)
    def _shift(j):
        sl = pl.ds(j * 16, 16)
        idx_v[sl] = idx_v[sl] >> 1

    for b in range(_NBUF):
        pltpu.async_copy(x_hbm.at[idx_v.at[pl.ds(b * _R, _R)]], bufs[b], gsem)

    @pl.loop(0, _NT, step=_NBUF)
    def _tile(t0):
        for b in range(_NBUF):
            t = t0 + b
            pltpu.make_async_copy(
                x_hbm.at[idx_v.at[pl.ds(0, _R)]], bufs[b], gsem).wait()
            pltpu.sync_copy(bufs[b], out_hbm.at[pl.ds(base + t * _R, _R)])
            nxt = t + _NBUF

            @pl.when(nxt < _NT)
            def _prefetch():
                pltpu.async_copy(
                    x_hbm.at[idx_v.at[pl.ds(nxt * _R, _R)]], bufs[b], gsem)


@functools.partial(jax.jit, static_argnames=())
def _dispatch(x, indices_s1, s1exp_to_s2_indices):
    mesh = plsc.VectorSubcoreMesh(core_axis_name="c", subcore_axis_name="s")
    f = pl.kernel(
        _dispatch_body,
        out_type=jax.ShapeDtypeStruct((_B, _D), jnp.float32),
        mesh=mesh,
        scratch_types=[
            pltpu.VMEM((_PER_W,), jnp.int32),
            pltpu.VMEM((_PER_W,), jnp.int32),
            pltpu.VMEM((_R, _D), jnp.float32),
            pltpu.VMEM((_R, _D), jnp.float32),
            pltpu.SemaphoreType.DMA,
            pltpu.SemaphoreType.DMA,
        ],
    )
    return f(x, indices_s1, s1exp_to_s2_indices)


def kernel(x, n_tokens, indices_s1, bin_ids_s1, bins_s1, ins_s1, outs_s1,
           ins_s2_virtual, outs_s2_virtual, s1exp_to_s2_indices, bin_ids_s2,
           bins_s2, ins_s2, outs_s2, tokens_per_expert_s1_exp,
           tokens_per_experts_s2_exp):
    return _dispatch(x, indices_s1, s1exp_to_s2_indices)
